# Initial kernel scaffold; baseline (speedup 1.0000x reference)
import jax
import jax.numpy as jnp
from jax.experimental import pallas as pl


def kernel(input, weights):
    raise NotImplementedError("write your pallas kernel here")


# speedup vs baseline: 1.9175x; 1.9175x over previous
import functools

import jax
import jax.numpy as jnp
from jax import lax
from jax.experimental import pallas as pl
from jax.experimental.pallas import tpu as pltpu
from jax.experimental.pallas import tpu_sc as plsc

PAD = 1
L = 16


def _positions(inp):
    mask = jnp.where(inp != PAD, 1, 0).astype(jnp.int32)
    return jnp.cumsum(mask, axis=1) * mask + PAD


def _sc_gather(positions_flat, weights, bsz, seq, d):
    NC, NS = 2, 16
    NW = NC * NS
    n = bsz * seq
    sl = n // NW
    G = 32
    ng = sl // G

    mesh = plsc.VectorSubcoreMesh(core_axis_name="c", subcore_axis_name="s")

    @functools.partial(
        pl.kernel,
        out_type=jax.ShapeDtypeStruct((n, d), jnp.float32),
        mesh=mesh,
        scratch_types=[
            pltpu.VMEM((sl,), jnp.int32),
            pltpu.VMEM((G, d), jnp.float32),
            pltpu.SemaphoreType.DMA,
            pltpu.SemaphoreType.DMA,
        ],
    )
    def k(idx_hbm, tab_hbm, out_hbm, idx_v, rows_v, sg, so):
        wid = lax.axis_index("s") * NC + lax.axis_index("c")
        base = wid * sl
        pltpu.sync_copy(idx_hbm.at[pl.ds(base, sl)], idx_v)

        def g_body(g, carry):
            pltpu.async_copy(
                tab_hbm.at[idx_v.at[pl.ds(g * G, G)]], rows_v, sg
            ).wait()
            pltpu.async_copy(
                rows_v, out_hbm.at[pl.ds(base + g * G, G)], so
            ).wait()
            return carry

        lax.fori_loop(0, ng, g_body, 0)

    return k(positions_flat, weights)


def kernel(input, weights):
    bsz, seq = input.shape
    nrows, d = weights.shape
    pos = _positions(input).reshape(bsz * seq)
    out = _sc_gather(pos, weights, bsz, seq, d)
    return lax.stop_gradient(out.reshape(bsz, seq, d))


# speedup vs baseline: 2.2227x; 1.1592x over previous
import functools

import jax
import jax.numpy as jnp
from jax import lax
from jax.experimental import pallas as pl
from jax.experimental.pallas import tpu as pltpu
from jax.experimental.pallas import tpu_sc as plsc

PAD = 1


def _pos_body(inp_ref, out_ref):
    x = inp_ref[...]
    m = jnp.where(x != PAD, 1, 0)
    b, s = x.shape
    cs = m
    k = 1
    while k < s:
        z = jnp.zeros((b, k), jnp.int32)
        cs = cs + jnp.concatenate([z, cs[:, : s - k]], axis=1)
        k *= 2
    out_ref[...] = cs * m + PAD


def _positions(inp):
    return pl.pallas_call(
        _pos_body,
        out_shape=jax.ShapeDtypeStruct(inp.shape, jnp.int32),
    )(inp)


def _sc_gather(positions_flat, weights, n, d):
    NC, NS = 2, 16
    NW = NC * NS
    sl = n // NW
    G = 32
    ng = sl // G

    mesh = plsc.VectorSubcoreMesh(core_axis_name="c", subcore_axis_name="s")

    @functools.partial(
        pl.kernel,
        out_type=jax.ShapeDtypeStruct((n, d), jnp.float32),
        mesh=mesh,
        scratch_types=[
            pltpu.VMEM((sl,), jnp.int32),
            pltpu.VMEM((2, G, d), jnp.float32),
            pltpu.SemaphoreType.DMA,
            pltpu.SemaphoreType.DMA,
            pltpu.SemaphoreType.DMA,
            pltpu.SemaphoreType.DMA,
        ],
    )
    def k(idx_hbm, tab_hbm, out_hbm, idx_v, rows_v, sg0, sg1, so0, so1):
        wid = lax.axis_index("s") * NC + lax.axis_index("c")
        base = wid * sl
        pltpu.sync_copy(idx_hbm.at[pl.ds(base, sl)], idx_v)

        r0 = rows_v.at[0]
        r1 = rows_v.at[1]

        def gath(g, r, sem):
            pltpu.async_copy(tab_hbm.at[idx_v.at[pl.ds(g * G, G)]], r, sem)

        def outw(g, r, sem):
            pltpu.async_copy(r, out_hbm.at[pl.ds(base + g * G, G)], sem)

        def wait_g(r, sem):
            pltpu.make_async_copy(tab_hbm.at[pl.ds(0, G)], r, sem).wait()

        def wait_o(r, sem):
            pltpu.make_async_copy(r, out_hbm.at[pl.ds(base, G)], sem).wait()

        gath(0, r0, sg0)
        gath(1, r1, sg1)

        def body(h, carry):
            g = 2 * h
            wait_g(r0, sg0)
            outw(g, r0, so0)
            wait_g(r1, sg1)
            outw(g + 1, r1, so1)
            wait_o(r0, so0)
            gath(g + 2, r0, sg0)
            wait_o(r1, so1)
            gath(g + 3, r1, sg1)
            return carry

        lax.fori_loop(0, ng // 2 - 1, body, 0)

        g = ng - 2
        wait_g(r0, sg0)
        outw(g, r0, so0)
        wait_g(r1, sg1)
        outw(g + 1, r1, so1)
        wait_o(r0, so0)
        wait_o(r1, so1)

    return k(positions_flat, weights)


def kernel(input, weights):
    bsz, seq = input.shape
    nrows, d = weights.shape
    pos = _positions(input).reshape(bsz * seq)
    out = _sc_gather(pos, weights, bsz * seq, d)
    return lax.stop_gradient(out.reshape(bsz, seq, d))


# speedup vs baseline: 2.3366x; 1.0512x over previous
import functools

import jax
import jax.numpy as jnp
from jax import lax
from jax.experimental import pallas as pl
from jax.experimental.pallas import tpu as pltpu
from jax.experimental.pallas import tpu_sc as plsc

PAD = 1


def _pos_body(inp_ref, out_ref):
    x = inp_ref[...]
    m = jnp.where(x != PAD, 1, 0)
    b, s = x.shape
    cs = m
    k = 1
    while k < s:
        z = jnp.zeros((b, k), jnp.int32)
        cs = cs + jnp.concatenate([z, cs[:, : s - k]], axis=1)
        k *= 2
    out_ref[...] = cs * m + PAD


def _positions(inp):
    return pl.pallas_call(
        _pos_body,
        out_shape=jax.ShapeDtypeStruct(inp.shape, jnp.int32),
    )(inp)


def _sc_gather(positions_flat, weights, n, d):
    NC, NS = 2, 16
    NW = NC * NS
    sl = n // NW
    G = 32
    ng = sl // G

    mesh = plsc.VectorSubcoreMesh(core_axis_name="c", subcore_axis_name="s")

    @functools.partial(
        pl.kernel,
        out_type=jax.ShapeDtypeStruct((n, d), jnp.float32),
        mesh=mesh,
        scratch_types=[
            pltpu.VMEM((sl,), jnp.int32),
            pltpu.VMEM((3, G, d), jnp.float32),
            pltpu.SemaphoreType.DMA,
            pltpu.SemaphoreType.DMA,
            pltpu.SemaphoreType.DMA,
            pltpu.SemaphoreType.DMA,
            pltpu.SemaphoreType.DMA,
            pltpu.SemaphoreType.DMA,
        ],
    )
    def k(idx_hbm, tab_hbm, out_hbm, idx_v, rows_v,
          sg0, sg1, sg2, so0, so1, so2):
        wid = lax.axis_index("s") * NC + lax.axis_index("c")
        base = wid * sl
        pltpu.sync_copy(idx_hbm.at[pl.ds(base, sl)], idx_v)

        r = [rows_v.at[0], rows_v.at[1], rows_v.at[2]]
        sg = [sg0, sg1, sg2]
        so = [so0, so1, so2]

        def gath(g, j):
            pltpu.async_copy(tab_hbm.at[idx_v.at[pl.ds(g * G, G)]], r[j], sg[j])

        def outw(g, j):
            pltpu.async_copy(r[j], out_hbm.at[pl.ds(base + g * G, G)], so[j])

        def wait_g(j):
            pltpu.make_async_copy(tab_hbm.at[pl.ds(0, G)], r[j], sg[j]).wait()

        def wait_o(j):
            pltpu.make_async_copy(r[j], out_hbm.at[pl.ds(base, G)], so[j]).wait()

        for j in range(3):
            gath(j, j)

        def body(h, carry):
            g = 3 * h
            for j in range(3):
                wait_g(j)
                outw(g + j, j)
                wait_o(j)
                gath(g + j + 3, j)
            return carry

        lax.fori_loop(0, (ng - 5) // 3, body, 0)

        gtail = ng - 5
        for j in range(3):
            wait_g(j)
            outw(gtail + j, j)
            if j < 2:
                wait_o(j)
                gath(gtail + j + 3, j)
        for j in range(2):
            wait_g(j)
            outw(ng - 2 + j, j)
        for j in range(3):
            wait_o(j)

    return k(positions_flat, weights)


def kernel(input, weights):
    bsz, seq = input.shape
    nrows, d = weights.shape
    pos = _positions(input).reshape(bsz * seq)
    out = _sc_gather(pos, weights, bsz * seq, d)
    return lax.stop_gradient(out.reshape(bsz, seq, d))
